# baseline (speedup 1.0000x reference)
import functools

import jax
import jax.numpy as jnp
import numpy as np
from jax import lax
from jax.experimental import pallas as pl
from jax.experimental.pallas import tpu as pltpu
from jax.experimental.pallas import tpu_sc as plsc

_N = 10000
_E = 320000
_B = 64
_H = 128
_NP = 10240
_RPS = _NP // 16
_ECH = 2560
_CPS = _ECH // 16
_R64 = _CPS * 2
_NBLKS_SC = _R64 // 8
_EPAD = _ECH * 128 - _E
def _pack_dup(x):
    h = lax.bitcast_convert_type(x.astype(jnp.bfloat16), jnp.uint16).astype(jnp.uint32)
    return lax.bitcast_convert_type(h | (h << 16), jnp.float32)


def _pack_w(wrow):
    h = lax.bitcast_convert_type(wrow.astype(jnp.bfloat16),
                                 jnp.uint16).astype(jnp.uint32).reshape(4, 2, 16)
    lo = h[:, 0, :]
    hi = h[:, 1, :]
    return lax.bitcast_convert_type((lo | (hi << 16)).reshape(64), jnp.float32)


def _bcast_lane(vec, idx16):
    return lax.gather(
        vec, idx16[:, None],
        lax.GatherDimensionNumbers(offset_dims=(), collapsed_slice_dims=(0,),
                                   start_index_map=(0,)),
        slice_sizes=(1,),
        mode=lax.GatherScatterMode.PROMISE_IN_BOUNDS)


def _sc_body(u_hbm, v_hbm, d_hbm, wa_hbm,
             t1_hbm, t2_hbm, deg_hbm,
             table, deg_stage, wa_v, ustage, vstage, dstage,
             rows_v, deg_v, degred, sem_in, sem_sc):
    cc = lax.axis_index("c")
    ss = lax.axis_index("s")

    pltpu.sync_copy(wa_hbm.at[cc], wa_v)

    zero16 = jnp.zeros((16,), jnp.float32)

    base = ss * _R64

    def start_in(blk, buf):
        row0 = base + blk * 8
        pltpu.async_copy(u_hbm.at[pl.ds(row0, 8)], ustage.at[buf], sem_in.at[buf])
        pltpu.async_copy(v_hbm.at[pl.ds(row0, 8)], vstage.at[buf], sem_in.at[buf])
        pltpu.async_copy(d_hbm.at[pl.ds(row0, 8)], dstage.at[buf], sem_in.at[buf])

    def wait_in(buf):
        pltpu.make_async_copy(u_hbm.at[pl.ds(0, 8)], ustage.at[buf],
                              sem_in.at[buf]).wait()
        pltpu.make_async_copy(v_hbm.at[pl.ds(0, 8)], vstage.at[buf],
                              sem_in.at[buf]).wait()
        pltpu.make_async_copy(d_hbm.at[pl.ds(0, 8)], dstage.at[buf],
                              sem_in.at[buf]).wait()

    def wait_sc(rb):
        pltpu.make_async_copy(rows_v.at[rb], table.at[pl.ds(0, 64)],
                              sem_sc.at[rb]).wait()


    def zrow(r, c):
        for j in range(8):
            rows_v[0, r, pl.ds(16 * j, 16)] = zero16
            rows_v[1, r, pl.ds(16 * j, 16)] = zero16
        return c

    lax.fori_loop(0, 64, zrow, 0)
    for q in range(_RPS // 64):
        pltpu.sync_copy(rows_v.at[0], table.at[pl.ds(_RPS * ss + 64 * q, 64)])

    def zdeg(zi, zc):
        deg_v[pl.ds(zi * 16, 16)] = zero16
        return zc

    lax.fori_loop(0, _NP // 16, zdeg, 0)
    start_in(0, 0)
    plsc.subcore_barrier()

    bf = jnp.bfloat16
    w = []
    for t in range(4):
        sl = pl.ds(16 * t, 16)
        w.append((plsc.bitcast(wa_v[0, sl], bf),
                  plsc.bitcast(wa_v[1, sl], bf),
                  plsc.bitcast(wa_v[2, sl], bf)))
    zero32 = jnp.zeros((32,), bf)

    lane = lax.broadcasted_iota(jnp.int32, (16,), 0)
    one16 = jnp.ones((16,), jnp.float32)

    def do_chunk(bi, c, rb):
        @plsc.parallel_loop(0, 8, step=1)
        def quad_body(q):
            gb = (q >> 1) * 16
            uvec = ustage[bi, c, pl.ds(gb, 16)]
            vvec = vstage[bi, c, pl.ds(gb, 16)]
            e0 = q * 8
            for k in range(8):
                ee = jnp.full((16,), (e0 + k) & 15, jnp.int32)
                ub = plsc.bitcast(_bcast_lane(uvec, ee), bf)
                vb = plsc.bitcast(_bcast_lane(vvec, ee), bf)
                for t in range(4):
                    w0t, w1t, bat = w[t]
                    m = jnp.maximum(ub * w0t + (vb * w1t + bat), zero32)
                    alo, ahi = plsc.unpack(m, format=plsc.PackFormat.INTERLEAVED)
                    rows_v[rb, e0 + k, pl.ds(32 * t, 16)] = alo
                    rows_v[rb, e0 + k, pl.ds(32 * t + 16, 16)] = ahi

        def deg_body(t, c2):
            dvec = dstage[bi, c, pl.ds(t * 16, 16)]
            for k in range(16):
                plsc.addupdate_scatter(deg_v, [dvec], one16, mask=lane == k)
            return c2

        lax.fori_loop(0, 4, deg_body, 0)
        pltpu.async_copy(rows_v.at[rb], table.at[dstage.at[bi, c]],
                         sem_sc.at[rb], add=True)

    def pair_body(q, carry):
        for half in range(2):
            blk = q * 2 + half
            bi = half
            wait_in(bi)
            for c in range(8):
                rb = c & 1
                if c >= 2:
                    wait_sc(rb)
                else:
                    @pl.when(blk > 0)
                    def _():
                        wait_sc(rb)
                do_chunk(bi, c, rb)
                if c == 1:
                    @pl.when(blk < _NBLKS_SC - 1)
                    def _():
                        start_in(blk + 1, 1 - bi)
        return carry

    lax.fori_loop(0, _NBLKS_SC // 2, pair_body, 0)
    wait_sc(0)
    wait_sc(1)
    pltpu.sync_copy(deg_v, deg_stage.at[ss])
    plsc.subcore_barrier()

    rs = pl.ds(_RPS * ss, _RPS)

    @pl.when(cc == 0)
    def _():
        pltpu.sync_copy(table.at[rs], t1_hbm.at[rs])
        for q2 in range(_RPS // 128):
            nsl = pl.ds(_RPS * ss + 128 * q2, 128)
            pltpu.sync_copy(deg_stage.at[:, nsl], degred)

            def red_body(k, c):
                sl = pl.ds(16 * k, 16)
                acc = degred[0, sl]
                for r in range(1, 16):
                    acc = acc + degred[r, sl]
                degred[0, sl] = acc
                return c

            lax.fori_loop(0, 8, red_body, 0)
            pltpu.sync_copy(degred.at[0], deg_hbm.at[nsl])

    @pl.when(cc == 1)
    def _():
        pltpu.sync_copy(table.at[rs], t2_hbm.at[rs])


def _sc_tables(u_p, v_p, d_p, waS):
    mesh = plsc.VectorSubcoreMesh(core_axis_name="c", subcore_axis_name="s")
    f32 = jnp.float32
    return pl.kernel(
        _sc_body,
        out_type=[jax.ShapeDtypeStruct((_NP, _H), f32),
                  jax.ShapeDtypeStruct((_NP, _H), f32),
                  jax.ShapeDtypeStruct((_NP,), f32)],
        mesh=mesh,
        compiler_params=pltpu.CompilerParams(needs_layout_passes=False),
        scratch_types=[
            pltpu.VMEM_SHARED((_NP, _H), f32),
            pltpu.VMEM_SHARED((16, _NP), f32),
            pltpu.VMEM((3, 64), f32),
            pltpu.VMEM((2, 8, 64), f32),
            pltpu.VMEM((2, 8, 64), f32),
            pltpu.VMEM((2, 8, 64), jnp.int32),
            pltpu.VMEM((2, 64, _H), f32),
            pltpu.VMEM((_NP,), f32),
            pltpu.VMEM((16, 128), f32),
            pltpu.SemaphoreType.DMA((2,)),
            pltpu.SemaphoreType.DMA((2,)),
        ],
    )(u_p, v_p, d_p, waS)


_RB = 1000
_NBLK = _N // _RB


def _tc_body(t1, t2, deg3, batch3,
             b1, g1, be1, g2, be2, g3, be3,
             c1Wb, c1bb, c2Wb, c2bb,
             Wp0, bp0, Wp1, bp1, Wp2, bp2,
             np_out, gpool_out,
             gmax1, gsum1, gmax2, gsum2, cnt_s):
    i = pl.program_id(0)
    kbn = np.float32(1.0 / np.sqrt(1.0 + 1e-5))
    hp = lax.Precision.HIGHEST

    @pl.when(i == 0)
    def _():
        gmax1[...] = jnp.full((_B, _H), -jnp.inf, jnp.float32)
        gmax2[...] = jnp.full((_B, _H), -jnp.inf, jnp.float32)
        gsum1[...] = jnp.zeros((_B, _H), jnp.float32)
        gsum2[...] = jnp.zeros((_B, _H), jnp.float32)
        cnt_s[...] = jnp.zeros((_B, _H), jnp.float32)

    A1 = t1[...]
    A2 = t2[...]
    deg = deg3[0]
    md = jnp.maximum(deg, 1.0)

    c1 = jax.nn.relu((jnp.dot(A1, c1Wb[...], precision=hp) + deg * c1bb[...][None, :]) / md)
    h1 = jax.nn.relu(c1 * (g2[...] * kbn)[None, :] + be2[...][None, :])
    c2 = jax.nn.relu((jnp.dot(A2, c2Wb[...], precision=hp) + deg * c2bb[...][None, :]) / md)
    h2 = jax.nn.relu(c2 * (g3[...] * kbn)[None, :] + be3[...][None, :])
    h0row = jax.nn.relu(b1[...] * (g1[...] * kbn) + be1[...])

    np_out[...] = h0row[None, :] + h1 + h2

    bids = batch3[0]
    lo = jnp.min(bids)
    hi = jnp.max(bids)

    def seg_body(b, carry):
        m = bids == b
        mf = m.astype(jnp.float32)
        sel = pl.ds(b, 1)
        p1 = jnp.max(jnp.where(m, h1, -jnp.inf), axis=0, keepdims=True)
        p2 = jnp.max(jnp.where(m, h2, -jnp.inf), axis=0, keepdims=True)
        s1 = jnp.sum(h1 * mf, axis=0, keepdims=True)
        s2 = jnp.sum(h2 * mf, axis=0, keepdims=True)
        pc = jnp.sum(mf)
        gmax1[sel, :] = jnp.maximum(gmax1[sel, :], p1)
        gmax2[sel, :] = jnp.maximum(gmax2[sel, :], p2)
        gsum1[sel, :] = gsum1[sel, :] + s1
        gsum2[sel, :] = gsum2[sel, :] + s2
        cnt_s[sel, :] = cnt_s[sel, :] + pc
        return carry

    lax.fori_loop(lo, hi + 1, seg_body, 0)

    @pl.when(i == _NBLK - 1)
    def _():
        cnt = cnt_s[...][:, :1]
        mc = jnp.maximum(cnt, 1.0)
        g0max = jnp.where(cnt > 0, h0row[None, :] + jnp.zeros((_B, 1), jnp.float32),
                          -jnp.inf)
        g0mean = (cnt * h0row[None, :]) / mc
        gm1 = gsum1[...] / mc
        gm2 = gsum2[...] / mc
        out = (jnp.dot(g0max, Wp0[...][:_H, :], precision=hp)
               + jnp.dot(g0mean, Wp0[...][_H:, :], precision=hp) + bp0[...][None, :]
               + jnp.dot(gmax1[...], Wp1[...][:_H, :], precision=hp)
               + jnp.dot(gm1, Wp1[...][_H:, :], precision=hp) + bp1[...][None, :]
               + jnp.dot(gmax2[...], Wp2[...][:_H, :], precision=hp)
               + jnp.dot(gm2, Wp2[...][_H:, :], precision=hp) + bp2[...][None, :])
        gpool_out[...] = out


def _tc_net(t1, t2, deg3, batch3, b1, g1, be1, g2, be2, g3, be3,
            c1Wb, c1bb, c2Wb, c2bb, Wp0, bp0, Wp1, bp1, Wp2, bp2):
    f32 = jnp.float32
    row_spec = pl.BlockSpec((_RB, _H), lambda i: (i, 0))
    small_spec = pl.BlockSpec((1, _RB, 1), lambda i: (i, 0, 0))
    full = lambda *shape: pl.BlockSpec(shape, lambda i: tuple(0 for _ in shape))
    vec = full(_H)
    mat = full(_H, _H)
    pmat = full(2 * _H, _H)
    return pl.pallas_call(
        _tc_body,
        grid=(_NBLK,),
        in_specs=[row_spec, row_spec, small_spec, small_spec,
                  vec, vec, vec, vec, vec, vec, vec,
                  mat, vec, mat, vec,
                  pmat, vec, pmat, vec, pmat, vec],
        out_specs=[pl.BlockSpec((_RB, _H), lambda i: (i, 0)),
                   pl.BlockSpec((_B, _H), lambda i: (0, 0))],
        out_shape=[jax.ShapeDtypeStruct((_N, _H), f32),
                   jax.ShapeDtypeStruct((_B, _H), f32)],
        scratch_shapes=[pltpu.VMEM((_B, _H), f32)] * 5,
    )(t1, t2, deg3, batch3, b1, g1, be1, g2, be2, g3, be3,
      c1Wb, c1bb, c2Wb, c2bb, Wp0, bp0, Wp1, bp1, Wp2, bp2)


def kernel(x, edge_index, edge_attr, batch, W1, b1, g1, be1, g2, be2, g3, be3,
           c1Wa, c1ba, c1Wb, c1bb, c2Wa, c2ba, c2Wb, c2bb,
           Wp0, bp0, Wp1, bp1, Wp2, bp2):
    f32 = jnp.float32
    dst = edge_index[1]
    u = edge_attr[:, 0]
    v = edge_attr[:, 1]
    zpad = jnp.zeros((_EPAD,), f32)
    u_p = _pack_dup(jnp.concatenate([u, zpad])).reshape(_ECH * 2, 64)
    v_p = _pack_dup(jnp.concatenate([v, zpad])).reshape(_ECH * 2, 64)
    dpad = (_N + (jnp.arange(_EPAD, dtype=jnp.int32) % 240))
    d_p = jnp.concatenate([dst, dpad]).reshape(_ECH * 2, 64)
    waS = jnp.stack([
        jnp.stack([_pack_w(c1Wa[0]), _pack_w(c1Wa[1]), _pack_w(c1ba)]),
        jnp.stack([_pack_w(c2Wa[0]), _pack_w(c2Wa[1]), _pack_w(c2ba)]),
    ])
    t1, t2, deg = _sc_tables(u_p, v_p, d_p, waS)

    batch3 = batch.reshape(_NBLK, _RB, 1)
    deg3 = deg[:_N].reshape(_NBLK, _RB, 1)
    node_pool, gpool = _tc_net(
        t1, t2, deg3, batch3, b1, g1, be1, g2, be2, g3, be3,
        c1Wb, c1bb, c2Wb, c2bb, Wp0, bp0, Wp1, bp1, Wp2, bp2)
    return (node_pool, gpool)


# speedup vs baseline: 1.3468x; 1.3468x over previous
import functools

import jax
import jax.numpy as jnp
import numpy as np
from jax import lax
from jax.experimental import pallas as pl
from jax.experimental.pallas import tpu as pltpu
from jax.experimental.pallas import tpu_sc as plsc

_N = 10000
_E = 320000
_B = 64
_H = 128
_NP = 10240
_RPS = _NP // 16
_ECH = 2560
_CPS = _ECH // 16
_R64 = _CPS * 2
_NBLKS_SC = _R64 // 8
_EPAD = _ECH * 128 - _E
def _pack_dup(x):
    h = lax.bitcast_convert_type(x.astype(jnp.bfloat16), jnp.uint16).astype(jnp.uint32)
    return lax.bitcast_convert_type(h | (h << 16), jnp.float32)


def _pack_w(wrow):
    h = lax.bitcast_convert_type(wrow.astype(jnp.bfloat16),
                                 jnp.uint16).astype(jnp.uint32).reshape(4, 2, 16)
    lo = h[:, 0, :]
    hi = h[:, 1, :]
    return lax.bitcast_convert_type((lo | (hi << 16)).reshape(64), jnp.float32)


def _bcast_lane(vec, idx16):
    return lax.gather(
        vec, idx16[:, None],
        lax.GatherDimensionNumbers(offset_dims=(), collapsed_slice_dims=(0,),
                                   start_index_map=(0,)),
        slice_sizes=(1,),
        mode=lax.GatherScatterMode.PROMISE_IN_BOUNDS)


def _sc_body(u_hbm, v_hbm, d_hbm, wa_hbm,
             t1_hbm, t2_hbm, deg_hbm,
             table, deg_stage, wa_v, ustage, vstage, dstage,
             rows_v, deg_v, degred, sem_in, sem_sc):
    cc = lax.axis_index("c")
    ss = lax.axis_index("s")

    pltpu.sync_copy(wa_hbm.at[cc], wa_v)

    zero16 = jnp.zeros((16,), jnp.float32)

    base = ss * _R64

    def start_in(blk, buf):
        row0 = base + blk * 8
        pltpu.async_copy(u_hbm.at[pl.ds(row0, 8)], ustage.at[buf], sem_in.at[buf])
        pltpu.async_copy(v_hbm.at[pl.ds(row0, 8)], vstage.at[buf], sem_in.at[buf])
        pltpu.async_copy(d_hbm.at[pl.ds(row0, 8)], dstage.at[buf], sem_in.at[buf])

    def wait_in(buf):
        pltpu.make_async_copy(u_hbm.at[pl.ds(0, 8)], ustage.at[buf],
                              sem_in.at[buf]).wait()
        pltpu.make_async_copy(v_hbm.at[pl.ds(0, 8)], vstage.at[buf],
                              sem_in.at[buf]).wait()
        pltpu.make_async_copy(d_hbm.at[pl.ds(0, 8)], dstage.at[buf],
                              sem_in.at[buf]).wait()

    def wait_sc(rb):
        pltpu.make_async_copy(rows_v.at[rb], table.at[pl.ds(0, 64)],
                              sem_sc.at[rb]).wait()


    def zrow(r, c):
        for j in range(8):
            rows_v[0, r, pl.ds(16 * j, 16)] = zero16
            rows_v[1, r, pl.ds(16 * j, 16)] = zero16
        return c

    lax.fori_loop(0, 64, zrow, 0)
    for q in range(_RPS // 64):
        pltpu.sync_copy(rows_v.at[0], table.at[pl.ds(_RPS * ss + 64 * q, 64)])

    def zdeg(zi, zc):
        deg_v[pl.ds(zi * 16, 16)] = zero16
        return zc

    lax.fori_loop(0, _NP // 16, zdeg, 0)
    start_in(0, 0)
    plsc.subcore_barrier()

    bf = jnp.bfloat16
    w = []
    for t in range(4):
        sl = pl.ds(16 * t, 16)
        w.append((plsc.bitcast(wa_v[0, sl], bf),
                  plsc.bitcast(wa_v[1, sl], bf),
                  plsc.bitcast(wa_v[2, sl], bf)))
    zero32 = jnp.zeros((32,), bf)

    lane = lax.broadcasted_iota(jnp.int32, (16,), 0)
    one16 = jnp.ones((16,), jnp.float32)

    def do_chunk(bi, c, rb):
        @plsc.parallel_loop(0, 16, step=1)
        def quad_body(q):
            gb = (q >> 2) * 16
            uvec = ustage[bi, c, pl.ds(gb, 16)]
            vvec = vstage[bi, c, pl.ds(gb, 16)]
            e0 = q * 4
            for k in range(4):
                ee = jnp.full((16,), (e0 + k) & 15, jnp.int32)
                ub = plsc.bitcast(_bcast_lane(uvec, ee), bf)
                vb = plsc.bitcast(_bcast_lane(vvec, ee), bf)
                for t in range(4):
                    w0t, w1t, bat = w[t]
                    m = jnp.maximum(ub * w0t + (vb * w1t + bat), zero32)
                    alo, ahi = plsc.unpack(m, format=plsc.PackFormat.INTERLEAVED)
                    rows_v[rb, e0 + k, pl.ds(32 * t, 16)] = alo
                    rows_v[rb, e0 + k, pl.ds(32 * t + 16, 16)] = ahi

        for t in range(4):
            dvec = dstage[bi, c, pl.ds(t * 16, 16)]
            plsc.addupdate_scatter(deg_v, [dvec], one16)
        pltpu.async_copy(rows_v.at[rb], table.at[dstage.at[bi, c]],
                         sem_sc.at[rb], add=True)

    def pair_body(q, carry):
        for half in range(2):
            blk = q * 2 + half
            bi = half
            wait_in(bi)
            for c in range(8):
                rb = c & 1
                if c >= 2:
                    wait_sc(rb)
                else:
                    @pl.when(blk > 0)
                    def _():
                        wait_sc(rb)
                do_chunk(bi, c, rb)
                if c == 1:
                    @pl.when(blk < _NBLKS_SC - 1)
                    def _():
                        start_in(blk + 1, 1 - bi)
        return carry

    lax.fori_loop(0, _NBLKS_SC // 2, pair_body, 0)
    wait_sc(0)
    wait_sc(1)
    pltpu.sync_copy(deg_v, deg_stage.at[ss])
    plsc.subcore_barrier()

    rs = pl.ds(_RPS * ss, _RPS)

    @pl.when(cc == 0)
    def _():
        pltpu.sync_copy(table.at[rs], t1_hbm.at[rs])
        for q2 in range(_RPS // 128):
            nsl = pl.ds(_RPS * ss + 128 * q2, 128)
            pltpu.sync_copy(deg_stage.at[:, nsl], degred)

            def red_body(k, c):
                sl = pl.ds(16 * k, 16)
                acc = degred[0, sl]
                for r in range(1, 16):
                    acc = acc + degred[r, sl]
                degred[0, sl] = acc
                return c

            lax.fori_loop(0, 8, red_body, 0)
            pltpu.sync_copy(degred.at[0], deg_hbm.at[nsl])

    @pl.when(cc == 1)
    def _():
        pltpu.sync_copy(table.at[rs], t2_hbm.at[rs])


def _sc_tables(u_p, v_p, d_p, waS):
    mesh = plsc.VectorSubcoreMesh(core_axis_name="c", subcore_axis_name="s")
    f32 = jnp.float32
    return pl.kernel(
        _sc_body,
        out_type=[jax.ShapeDtypeStruct((_NP, _H), f32),
                  jax.ShapeDtypeStruct((_NP, _H), f32),
                  jax.ShapeDtypeStruct((_NP,), f32)],
        mesh=mesh,
        compiler_params=pltpu.CompilerParams(needs_layout_passes=False),
        scratch_types=[
            pltpu.VMEM_SHARED((_NP, _H), f32),
            pltpu.VMEM_SHARED((16, _NP), f32),
            pltpu.VMEM((3, 64), f32),
            pltpu.VMEM((2, 8, 64), f32),
            pltpu.VMEM((2, 8, 64), f32),
            pltpu.VMEM((2, 8, 64), jnp.int32),
            pltpu.VMEM((2, 64, _H), f32),
            pltpu.VMEM((_NP,), f32),
            pltpu.VMEM((16, 128), f32),
            pltpu.SemaphoreType.DMA((2,)),
            pltpu.SemaphoreType.DMA((2,)),
        ],
    )(u_p, v_p, d_p, waS)


_RB = 1000
_NBLK = _N // _RB


def _tc_body(t1, t2, deg3, batch3,
             b1, g1, be1, g2, be2, g3, be3,
             c1Wb, c1bb, c2Wb, c2bb,
             Wp0, bp0, Wp1, bp1, Wp2, bp2,
             np_out, gpool_out,
             gmax1, gsum1, gmax2, gsum2, cnt_s):
    i = pl.program_id(0)
    kbn = np.float32(1.0 / np.sqrt(1.0 + 1e-5))
    hp = lax.Precision.HIGHEST

    @pl.when(i == 0)
    def _():
        gmax1[...] = jnp.full((_B, _H), -jnp.inf, jnp.float32)
        gmax2[...] = jnp.full((_B, _H), -jnp.inf, jnp.float32)
        gsum1[...] = jnp.zeros((_B, _H), jnp.float32)
        gsum2[...] = jnp.zeros((_B, _H), jnp.float32)
        cnt_s[...] = jnp.zeros((_B, _H), jnp.float32)

    A1 = t1[...]
    A2 = t2[...]
    deg = deg3[0]
    md = jnp.maximum(deg, 1.0)

    c1 = jax.nn.relu((jnp.dot(A1, c1Wb[...], precision=hp) + deg * c1bb[...][None, :]) / md)
    h1 = jax.nn.relu(c1 * (g2[...] * kbn)[None, :] + be2[...][None, :])
    c2 = jax.nn.relu((jnp.dot(A2, c2Wb[...], precision=hp) + deg * c2bb[...][None, :]) / md)
    h2 = jax.nn.relu(c2 * (g3[...] * kbn)[None, :] + be3[...][None, :])
    h0row = jax.nn.relu(b1[...] * (g1[...] * kbn) + be1[...])

    np_out[...] = h0row[None, :] + h1 + h2

    bids = batch3[0]
    lo = jnp.min(bids)
    hi = jnp.max(bids)

    def seg_body(b, carry):
        m = bids == b
        mf = m.astype(jnp.float32)
        sel = pl.ds(b, 1)
        p1 = jnp.max(jnp.where(m, h1, -jnp.inf), axis=0, keepdims=True)
        p2 = jnp.max(jnp.where(m, h2, -jnp.inf), axis=0, keepdims=True)
        s1 = jnp.sum(h1 * mf, axis=0, keepdims=True)
        s2 = jnp.sum(h2 * mf, axis=0, keepdims=True)
        pc = jnp.sum(mf)
        gmax1[sel, :] = jnp.maximum(gmax1[sel, :], p1)
        gmax2[sel, :] = jnp.maximum(gmax2[sel, :], p2)
        gsum1[sel, :] = gsum1[sel, :] + s1
        gsum2[sel, :] = gsum2[sel, :] + s2
        cnt_s[sel, :] = cnt_s[sel, :] + pc
        return carry

    lax.fori_loop(lo, hi + 1, seg_body, 0)

    @pl.when(i == _NBLK - 1)
    def _():
        cnt = cnt_s[...][:, :1]
        mc = jnp.maximum(cnt, 1.0)
        g0max = jnp.where(cnt > 0, h0row[None, :] + jnp.zeros((_B, 1), jnp.float32),
                          -jnp.inf)
        g0mean = (cnt * h0row[None, :]) / mc
        gm1 = gsum1[...] / mc
        gm2 = gsum2[...] / mc
        out = (jnp.dot(g0max, Wp0[...][:_H, :], precision=hp)
               + jnp.dot(g0mean, Wp0[...][_H:, :], precision=hp) + bp0[...][None, :]
               + jnp.dot(gmax1[...], Wp1[...][:_H, :], precision=hp)
               + jnp.dot(gm1, Wp1[...][_H:, :], precision=hp) + bp1[...][None, :]
               + jnp.dot(gmax2[...], Wp2[...][:_H, :], precision=hp)
               + jnp.dot(gm2, Wp2[...][_H:, :], precision=hp) + bp2[...][None, :])
        gpool_out[...] = out


def _tc_net(t1, t2, deg3, batch3, b1, g1, be1, g2, be2, g3, be3,
            c1Wb, c1bb, c2Wb, c2bb, Wp0, bp0, Wp1, bp1, Wp2, bp2):
    f32 = jnp.float32
    row_spec = pl.BlockSpec((_RB, _H), lambda i: (i, 0))
    small_spec = pl.BlockSpec((1, _RB, 1), lambda i: (i, 0, 0))
    full = lambda *shape: pl.BlockSpec(shape, lambda i: tuple(0 for _ in shape))
    vec = full(_H)
    mat = full(_H, _H)
    pmat = full(2 * _H, _H)
    return pl.pallas_call(
        _tc_body,
        grid=(_NBLK,),
        in_specs=[row_spec, row_spec, small_spec, small_spec,
                  vec, vec, vec, vec, vec, vec, vec,
                  mat, vec, mat, vec,
                  pmat, vec, pmat, vec, pmat, vec],
        out_specs=[pl.BlockSpec((_RB, _H), lambda i: (i, 0)),
                   pl.BlockSpec((_B, _H), lambda i: (0, 0))],
        out_shape=[jax.ShapeDtypeStruct((_N, _H), f32),
                   jax.ShapeDtypeStruct((_B, _H), f32)],
        scratch_shapes=[pltpu.VMEM((_B, _H), f32)] * 5,
    )(t1, t2, deg3, batch3, b1, g1, be1, g2, be2, g3, be3,
      c1Wb, c1bb, c2Wb, c2bb, Wp0, bp0, Wp1, bp1, Wp2, bp2)


def kernel(x, edge_index, edge_attr, batch, W1, b1, g1, be1, g2, be2, g3, be3,
           c1Wa, c1ba, c1Wb, c1bb, c2Wa, c2ba, c2Wb, c2bb,
           Wp0, bp0, Wp1, bp1, Wp2, bp2):
    f32 = jnp.float32
    dst = edge_index[1]
    u = edge_attr[:, 0]
    v = edge_attr[:, 1]
    zpad = jnp.zeros((_EPAD,), f32)
    u_p = _pack_dup(jnp.concatenate([u, zpad])).reshape(_ECH * 2, 64)
    v_p = _pack_dup(jnp.concatenate([v, zpad])).reshape(_ECH * 2, 64)
    dpad = (_N + (jnp.arange(_EPAD, dtype=jnp.int32) % 240))
    d_p = jnp.concatenate([dst, dpad]).reshape(_ECH * 2, 64)
    waS = jnp.stack([
        jnp.stack([_pack_w(c1Wa[0]), _pack_w(c1Wa[1]), _pack_w(c1ba)]),
        jnp.stack([_pack_w(c2Wa[0]), _pack_w(c2Wa[1]), _pack_w(c2ba)]),
    ])
    t1, t2, deg = _sc_tables(u_p, v_p, d_p, waS)

    batch3 = batch.reshape(_NBLK, _RB, 1)
    deg3 = deg[:_N].reshape(_NBLK, _RB, 1)
    node_pool, gpool = _tc_net(
        t1, t2, deg3, batch3, b1, g1, be1, g2, be2, g3, be3,
        c1Wb, c1bb, c2Wb, c2bb, Wp0, bp0, Wp1, bp1, Wp2, bp2)
    return (node_pool, gpool)


# speedup vs baseline: 1.3470x; 1.0002x over previous
import jax
import jax.numpy as jnp
import numpy as np
from jax import lax
from jax.experimental import pallas as pl
from jax.experimental.pallas import tpu as pltpu
from jax.experimental.pallas import tpu_sc as plsc

_N = 10000
_E = 320000
_B = 64
_H = 128
_NP = 10240
_RPS = _NP // 16
_ECH = 2560
_CPS = _ECH // 16
_R64 = _CPS * 2
_NBLKS_SC = _R64 // 8
_EPAD = _ECH * 128 - _E
def _pack_dup(x):
    h = lax.bitcast_convert_type(x.astype(jnp.bfloat16), jnp.uint16).astype(jnp.uint32)
    return lax.bitcast_convert_type(h | (h << 16), jnp.float32)


def _pack_w(wrow):
    h = lax.bitcast_convert_type(wrow.astype(jnp.bfloat16),
                                 jnp.uint16).astype(jnp.uint32).reshape(4, 2, 16)
    lo = h[:, 0, :]
    hi = h[:, 1, :]
    return lax.bitcast_convert_type((lo | (hi << 16)).reshape(64), jnp.float32)


def _bcast_lane(vec, idx16):
    return lax.gather(
        vec, idx16[:, None],
        lax.GatherDimensionNumbers(offset_dims=(), collapsed_slice_dims=(0,),
                                   start_index_map=(0,)),
        slice_sizes=(1,),
        mode=lax.GatherScatterMode.PROMISE_IN_BOUNDS)


def _sc_body(u_hbm, v_hbm, d_hbm, wa_hbm,
             t1_hbm, t2_hbm, deg_hbm,
             table, deg_stage, wa_v, ustage, vstage, dstage,
             rows_v, deg_v, degred, sem_in, sem_sc):
    cc = lax.axis_index("c")
    ss = lax.axis_index("s")

    pltpu.sync_copy(wa_hbm.at[cc], wa_v)

    zero16 = jnp.zeros((16,), jnp.float32)

    base = ss * _R64

    def start_in(blk, buf):
        row0 = base + blk * 8
        pltpu.async_copy(u_hbm.at[pl.ds(row0, 8)], ustage.at[buf], sem_in.at[buf])
        pltpu.async_copy(v_hbm.at[pl.ds(row0, 8)], vstage.at[buf], sem_in.at[buf])
        pltpu.async_copy(d_hbm.at[pl.ds(row0, 8)], dstage.at[buf], sem_in.at[buf])

    def wait_in(buf):
        pltpu.make_async_copy(u_hbm.at[pl.ds(0, 8)], ustage.at[buf],
                              sem_in.at[buf]).wait()
        pltpu.make_async_copy(v_hbm.at[pl.ds(0, 8)], vstage.at[buf],
                              sem_in.at[buf]).wait()
        pltpu.make_async_copy(d_hbm.at[pl.ds(0, 8)], dstage.at[buf],
                              sem_in.at[buf]).wait()

    def wait_sc(rb):
        pltpu.make_async_copy(rows_v.at[rb], table.at[pl.ds(0, 64)],
                              sem_sc.at[rb]).wait()


    def zrow(r, c):
        for j in range(8):
            rows_v[0, r, pl.ds(16 * j, 16)] = zero16
            rows_v[1, r, pl.ds(16 * j, 16)] = zero16
        return c

    lax.fori_loop(0, 64, zrow, 0)
    for q in range(_RPS // 64):
        pltpu.sync_copy(rows_v.at[0], table.at[pl.ds(_RPS * ss + 64 * q, 64)])

    def zdeg(zi, zc):
        deg_v[pl.ds(zi * 16, 16)] = zero16
        return zc

    lax.fori_loop(0, _NP // 16, zdeg, 0)
    start_in(0, 0)
    plsc.subcore_barrier()

    bf = jnp.bfloat16
    w = []
    for t in range(4):
        sl = pl.ds(16 * t, 16)
        w.append((plsc.bitcast(wa_v[0, sl], bf),
                  plsc.bitcast(wa_v[1, sl], bf),
                  plsc.bitcast(wa_v[2, sl], bf)))
    zero32 = jnp.zeros((32,), bf)

    lane = lax.broadcasted_iota(jnp.int32, (16,), 0)
    one16 = jnp.ones((16,), jnp.float32)

    def do_chunk(bi, c, rb):
        @plsc.parallel_loop(0, 16, step=1)
        def quad_body(q):
            gb = (q >> 2) * 16
            uvec = ustage[bi, c, pl.ds(gb, 16)]
            vvec = vstage[bi, c, pl.ds(gb, 16)]
            e0 = q * 4
            for k in range(4):
                ee = jnp.full((16,), (e0 + k) & 15, jnp.int32)
                ub = plsc.bitcast(_bcast_lane(uvec, ee), bf)
                vb = plsc.bitcast(_bcast_lane(vvec, ee), bf)
                for t in range(4):
                    w0t, w1t, bat = w[t]
                    m = jnp.maximum(ub * w0t + (vb * w1t + bat), zero32)
                    alo, ahi = plsc.unpack(m, format=plsc.PackFormat.INTERLEAVED)
                    rows_v[rb, e0 + k, pl.ds(32 * t, 16)] = alo
                    rows_v[rb, e0 + k, pl.ds(32 * t + 16, 16)] = ahi

        for t in range(4):
            dvec = dstage[bi, c, pl.ds(t * 16, 16)]
            plsc.addupdate_scatter(deg_v, [dvec], one16)
        pltpu.async_copy(rows_v.at[rb], table.at[dstage.at[bi, c]],
                         sem_sc.at[rb], add=True)

    def pair_body(q, carry):
        for half in range(2):
            blk = q * 2 + half
            bi = half
            wait_in(bi)
            for c in range(8):
                rb = c & 1
                if c >= 2:
                    wait_sc(rb)
                else:
                    @pl.when(blk > 0)
                    def _():
                        wait_sc(rb)
                do_chunk(bi, c, rb)
                if c == 1:
                    @pl.when(blk < _NBLKS_SC - 1)
                    def _():
                        start_in(blk + 1, 1 - bi)
        return carry

    lax.fori_loop(0, _NBLKS_SC // 2, pair_body, 0)
    wait_sc(0)
    wait_sc(1)
    pltpu.sync_copy(deg_v, deg_stage.at[ss])
    plsc.subcore_barrier()

    rs = pl.ds(_RPS * ss, _RPS)

    @pl.when(cc == 0)
    def _():
        pltpu.sync_copy(table.at[rs], t1_hbm.at[rs])
        for q2 in range(_RPS // 128):
            nsl = pl.ds(_RPS * ss + 128 * q2, 128)
            pltpu.sync_copy(deg_stage.at[:, nsl], degred)

            def red_body(k, c):
                sl = pl.ds(16 * k, 16)
                acc = degred[0, sl]
                for r in range(1, 16):
                    acc = acc + degred[r, sl]
                degred[0, sl] = acc
                return c

            lax.fori_loop(0, 8, red_body, 0)
            pltpu.sync_copy(degred.at[0], deg_hbm.at[nsl])

    @pl.when(cc == 1)
    def _():
        pltpu.sync_copy(table.at[rs], t2_hbm.at[rs])


def _sc_tables(u_p, v_p, d_p, waS):
    mesh = plsc.VectorSubcoreMesh(core_axis_name="c", subcore_axis_name="s")
    f32 = jnp.float32
    return pl.kernel(
        _sc_body,
        out_type=[jax.ShapeDtypeStruct((_NP, _H), f32),
                  jax.ShapeDtypeStruct((_NP, _H), f32),
                  jax.ShapeDtypeStruct((_NP,), f32)],
        mesh=mesh,
        compiler_params=pltpu.CompilerParams(needs_layout_passes=False),
        scratch_types=[
            pltpu.VMEM_SHARED((_NP, _H), f32),
            pltpu.VMEM_SHARED((16, _NP), f32),
            pltpu.VMEM((3, 64), f32),
            pltpu.VMEM((2, 8, 64), f32),
            pltpu.VMEM((2, 8, 64), f32),
            pltpu.VMEM((2, 8, 64), jnp.int32),
            pltpu.VMEM((2, 64, _H), f32),
            pltpu.VMEM((_NP,), f32),
            pltpu.VMEM((16, 128), f32),
            pltpu.SemaphoreType.DMA((2,)),
            pltpu.SemaphoreType.DMA((2,)),
        ],
    )(u_p, v_p, d_p, waS)


_RB = 1000
_NBLK = _N // _RB


def _tc_body(t1, t2, deg3, batch3,
             b1, g1, be1, g2, be2, g3, be3,
             c1Wb, c1bb, c2Wb, c2bb,
             Wp0, bp0, Wp1, bp1, Wp2, bp2,
             np_out, gpool_out,
             gmax1, gsum1, gmax2, gsum2, cnt_s):
    i = pl.program_id(0)
    kbn = np.float32(1.0 / np.sqrt(1.0 + 1e-5))
    hp = lax.Precision.HIGHEST

    @pl.when(i == 0)
    def _():
        gmax1[...] = jnp.full((_B, _H), -jnp.inf, jnp.float32)
        gmax2[...] = jnp.full((_B, _H), -jnp.inf, jnp.float32)
        gsum1[...] = jnp.zeros((_B, _H), jnp.float32)
        gsum2[...] = jnp.zeros((_B, _H), jnp.float32)
        cnt_s[...] = jnp.zeros((_B, _H), jnp.float32)

    A1 = t1[...]
    A2 = t2[...]
    deg = deg3[0]
    md = jnp.maximum(deg, 1.0)

    c1 = jax.nn.relu((jnp.dot(A1, c1Wb[...], precision=hp) + deg * c1bb[...][None, :]) / md)
    h1 = jax.nn.relu(c1 * (g2[...] * kbn)[None, :] + be2[...][None, :])
    c2 = jax.nn.relu((jnp.dot(A2, c2Wb[...], precision=hp) + deg * c2bb[...][None, :]) / md)
    h2 = jax.nn.relu(c2 * (g3[...] * kbn)[None, :] + be3[...][None, :])
    h0row = jax.nn.relu(b1[...] * (g1[...] * kbn) + be1[...])

    np_out[...] = h0row[None, :] + h1 + h2

    bids = batch3[0]
    lo = jnp.min(bids)
    hi = jnp.max(bids)

    def seg_body(b, carry):
        m = bids == b
        mf = m.astype(jnp.float32)
        sel = pl.ds(b, 1)
        p1 = jnp.max(jnp.where(m, h1, -jnp.inf), axis=0, keepdims=True)
        p2 = jnp.max(jnp.where(m, h2, -jnp.inf), axis=0, keepdims=True)
        s1 = jnp.sum(h1 * mf, axis=0, keepdims=True)
        s2 = jnp.sum(h2 * mf, axis=0, keepdims=True)
        pc = jnp.sum(mf)
        gmax1[sel, :] = jnp.maximum(gmax1[sel, :], p1)
        gmax2[sel, :] = jnp.maximum(gmax2[sel, :], p2)
        gsum1[sel, :] = gsum1[sel, :] + s1
        gsum2[sel, :] = gsum2[sel, :] + s2
        cnt_s[sel, :] = cnt_s[sel, :] + pc
        return carry

    lax.fori_loop(lo, hi + 1, seg_body, 0)

    @pl.when(i == _NBLK - 1)
    def _():
        cnt = cnt_s[...][:, :1]
        mc = jnp.maximum(cnt, 1.0)
        g0max = jnp.where(cnt > 0, h0row[None, :] + jnp.zeros((_B, 1), jnp.float32),
                          -jnp.inf)
        g0mean = (cnt * h0row[None, :]) / mc
        gm1 = gsum1[...] / mc
        gm2 = gsum2[...] / mc
        out = (jnp.dot(g0max, Wp0[...][:_H, :], precision=hp)
               + jnp.dot(g0mean, Wp0[...][_H:, :], precision=hp) + bp0[...][None, :]
               + jnp.dot(gmax1[...], Wp1[...][:_H, :], precision=hp)
               + jnp.dot(gm1, Wp1[...][_H:, :], precision=hp) + bp1[...][None, :]
               + jnp.dot(gmax2[...], Wp2[...][:_H, :], precision=hp)
               + jnp.dot(gm2, Wp2[...][_H:, :], precision=hp) + bp2[...][None, :])
        gpool_out[...] = out


def _tc_net(t1, t2, deg3, batch3, b1, g1, be1, g2, be2, g3, be3,
            c1Wb, c1bb, c2Wb, c2bb, Wp0, bp0, Wp1, bp1, Wp2, bp2):
    f32 = jnp.float32
    row_spec = pl.BlockSpec((_RB, _H), lambda i: (i, 0))
    small_spec = pl.BlockSpec((1, _RB, 1), lambda i: (i, 0, 0))
    full = lambda *shape: pl.BlockSpec(shape, lambda i: tuple(0 for _ in shape))
    vec = full(_H)
    mat = full(_H, _H)
    pmat = full(2 * _H, _H)
    return pl.pallas_call(
        _tc_body,
        grid=(_NBLK,),
        in_specs=[row_spec, row_spec, small_spec, small_spec,
                  vec, vec, vec, vec, vec, vec, vec,
                  mat, vec, mat, vec,
                  pmat, vec, pmat, vec, pmat, vec],
        out_specs=[pl.BlockSpec((_RB, _H), lambda i: (i, 0)),
                   pl.BlockSpec((_B, _H), lambda i: (0, 0))],
        out_shape=[jax.ShapeDtypeStruct((_N, _H), f32),
                   jax.ShapeDtypeStruct((_B, _H), f32)],
        scratch_shapes=[pltpu.VMEM((_B, _H), f32)] * 5,
    )(t1, t2, deg3, batch3, b1, g1, be1, g2, be2, g3, be3,
      c1Wb, c1bb, c2Wb, c2bb, Wp0, bp0, Wp1, bp1, Wp2, bp2)


def kernel(x, edge_index, edge_attr, batch, W1, b1, g1, be1, g2, be2, g3, be3,
           c1Wa, c1ba, c1Wb, c1bb, c2Wa, c2ba, c2Wb, c2bb,
           Wp0, bp0, Wp1, bp1, Wp2, bp2):
    f32 = jnp.float32
    dst = edge_index[1]
    u = edge_attr[:, 0]
    v = edge_attr[:, 1]
    zpad = jnp.zeros((_EPAD,), f32)
    u_p = _pack_dup(jnp.concatenate([u, zpad])).reshape(_ECH * 2, 64)
    v_p = _pack_dup(jnp.concatenate([v, zpad])).reshape(_ECH * 2, 64)
    dpad = (_N + (jnp.arange(_EPAD, dtype=jnp.int32) % 240))
    d_p = jnp.concatenate([dst, dpad]).reshape(_ECH * 2, 64)
    waS = jnp.stack([
        jnp.stack([_pack_w(c1Wa[0]), _pack_w(c1Wa[1]), _pack_w(c1ba)]),
        jnp.stack([_pack_w(c2Wa[0]), _pack_w(c2Wa[1]), _pack_w(c2ba)]),
    ])
    t1, t2, deg = _sc_tables(u_p, v_p, d_p, waS)

    batch3 = batch.reshape(_NBLK, _RB, 1)
    deg3 = deg[:_N].reshape(_NBLK, _RB, 1)
    node_pool, gpool = _tc_net(
        t1, t2, deg3, batch3, b1, g1, be1, g2, be2, g3, be3,
        c1Wb, c1bb, c2Wb, c2bb, Wp0, bp0, Wp1, bp1, Wp2, bp2)
    return (node_pool, gpool)
